# baseline (speedup 1.0000x reference)
import functools

import jax
import jax.numpy as jnp
from jax import lax
from jax.experimental import pallas as pl
from jax.experimental.pallas import tpu as pltpu
from jax.experimental.pallas import tpu_sc as plsc

B = 4096
L = 50
HID = 128
NW = 32
ROWS_W = B // NW
NBUF = 5
AHEAD = 3
HALF = ROWS_W // 2
NC = 2 * L


def _body(eeg_tab, ecg_tab, eeg_idx, ecg_idx, eeg_out, ecg_out,
          idx_v, *ring):
    bufs = ring[:NBUF]
    gsem = ring[NBUF:2 * NBUF]
    ssem = ring[2 * NBUF:]
    wid = lax.axis_index("c") * 16 + lax.axis_index("s")
    row_base = wid * ROWS_W

    for (tab, idx_hbm, out_hbm) in (
        (eeg_tab, eeg_idx, eeg_out),
        (ecg_tab, ecg_idx, ecg_out),
    ):
        pltpu.sync_copy(idx_hbm.at[:, pl.ds(row_base, ROWS_W)], idx_v)

        def gather(c, b):
            return pltpu.make_async_copy(
                tab.at[idx_v.at[c // 2, pl.ds((c % 2) * HALF, HALF)]],
                bufs[b], gsem[b])

        def store(c, b):
            return pltpu.make_async_copy(
                bufs[b],
                out_hbm.at[c // 2, pl.ds(row_base + (c % 2) * HALF, HALF)],
                ssem[b])

        for c in range(AHEAD):
            gather(c, c).start()

        def turn(jj, _):
            for b in range(NBUF):
                cj = jj * NBUF + b
                @pl.when(cj >= NBUF - AHEAD)
                def _drain():
                    store(cj - (NBUF - AHEAD), (b + AHEAD) % NBUF).wait()

                @pl.when(cj + AHEAD < NC)
                def _fire():
                    gather(cj + AHEAD, (b + AHEAD) % NBUF).start()

                gather(cj, b).wait()
                store(cj, b).start()
            return _

        lax.fori_loop(0, NC // NBUF, turn, None)

        for c in range(NC - (NBUF - AHEAD), NC):
            store(c, c % NBUF).wait()


@functools.partial(jax.jit, static_argnums=())
def kernel(eeg_input_ids, ecg_input_ids, eeg_table, ecg_table):
    eeg_idx = eeg_input_ids.astype(jnp.int32).T
    ecg_idx = ecg_input_ids.astype(jnp.int32).T

    mesh = plsc.VectorSubcoreMesh(core_axis_name="c", subcore_axis_name="s")
    run = pl.kernel(
        _body,
        mesh=mesh,
        out_type=[
            jax.ShapeDtypeStruct((L, B, HID), jnp.float32),
            jax.ShapeDtypeStruct((L, B, HID), jnp.float32),
        ],
        scratch_types=(
            [pltpu.VMEM((L, ROWS_W), jnp.int32)]
            + [pltpu.VMEM((HALF, HID), jnp.float32)] * NBUF
            + [pltpu.SemaphoreType.DMA] * (2 * NBUF)
        ),
    )
    eeg_t, ecg_t = run(eeg_table, ecg_table, eeg_idx, ecg_idx)
    return (eeg_t.transpose(1, 0, 2), ecg_t.transpose(1, 0, 2))
